# baseline (speedup 1.0000x reference)
import functools

import jax
import jax.numpy as jnp
from jax import lax
from jax.experimental import pallas as pl
from jax.experimental.pallas import tpu as pltpu
from jax.experimental.pallas import tpu_sc as plsc

N = 100000
E = 6400000
G = 64
NCLS = 10

NSC = 2
NTILE = 16
GRP = 128
WG = 16
W = GRP * WG
E_PAD = 6422528
WINDOWS = E_PAD // (NSC * NTILE * W)
EG_ROWS = E_PAD // GRP
PWG = 4
PW = GRP * PWG
PWIN = E_PAD // (NSC * NTILE * PW)

NP1 = 100008
NACC = 100096
ACC_T = 6256
NDEG = 100096
DEG_T = 6256
SLICESZ = tuple((k * 512, 512) for k in range(12)) + ((6144, 112),)

ZROWS = 1024


def _zero_loop(zbuf, nrows, width):
    zv = jnp.zeros((16,), jnp.float32)

    def body(i, _):
        r = i // (width // 16)
        k = i % (width // 16)
        zbuf[r, pl.ds(k * 16, 16)] = zv
        return _

    lax.fori_loop(0, nrows * (width // 16), body, 0)


def _deg_body(dst2, out0, out1, acc, dstbuf, ones_v, zbuf, ssem):
    core = lax.axis_index("c")
    tile = lax.axis_index("s")

    zv = jnp.zeros((16,), jnp.float32)
    ov = jnp.ones((16,), jnp.float32)

    def stage(i, _):
        zbuf[pl.ds(i * 16, 16)] = zv
        return _

    lax.fori_loop(0, DEG_T // 16, stage, 0)

    def ones_body(i, _):
        ones_v[pl.ds(i * 16, 16)] = ov
        return _

    lax.fori_loop(0, GRP // 16, ones_body, 0)

    base = tile * DEG_T
    pltpu.sync_copy(zbuf.at[pl.ds(0, DEG_T)], acc.at[pl.ds(base, DEG_T)])
    plsc.subcore_barrier()

    gbase = (core * NTILE + tile) * (WINDOWS * WG)

    def window(w, _):
        gw = gbase + w * WG
        pltpu.sync_copy(dst2.at[pl.ds(gw, WG)], dstbuf)

        def fire(j, _):
            pltpu.async_copy(ones_v, acc.at[dstbuf.at[j]], ssem, add=True)
            return _

        lax.fori_loop(0, WG, fire, 0)

        def drain(j, _):
            pltpu.make_async_copy(ones_v, acc.at[dstbuf.at[j]], ssem).wait()
            return _

        lax.fori_loop(0, WG, drain, 0)
        return _

    lax.fori_loop(0, WINDOWS, window, 0)
    plsc.subcore_barrier()
    pltpu.sync_copy(acc.at[pl.ds(base, DEG_T)], zbuf)
    for k, o in enumerate((out0, out1)):
        @pl.when(core == k)
        def _copy(o=o):
            pltpu.sync_copy(zbuf, o.at[pl.ds(base, DEG_T)])


def _make_deg_kernel():
    mesh = plsc.VectorSubcoreMesh(core_axis_name="c", subcore_axis_name="s")
    return pl.kernel(
        _deg_body,
        out_type=[jax.ShapeDtypeStruct((NDEG,), jnp.float32),
                  jax.ShapeDtypeStruct((NDEG,), jnp.float32)],
        mesh=mesh,
        scratch_types=[
            pltpu.VMEM_SHARED((NDEG,), jnp.float32),
            pltpu.VMEM((WG, GRP), jnp.int32),
            pltpu.VMEM((GRP,), jnp.float32),
            pltpu.VMEM((DEG_T,), jnp.float32),
            pltpu.SemaphoreType.DMA,
        ],
    )


def _prop_body(nchunk, yflat, src1, dst2, out, acc, srcbuf, dstbuf, rows,
               zbuf, isem, gsem, ssem):
    core = lax.axis_index("c")
    tile = lax.axis_index("s")
    gbase = (core * NTILE + tile) * (PWIN * PWG)
    zbase = tile * ACC_T

    def issue_idx(w):
        eb = gbase * GRP + w * PW
        gw = gbase + w * PWG
        pltpu.async_copy(src1.at[pl.ds(eb, PW)], srcbuf.at[lax.rem(w, 2)], isem)
        pltpu.async_copy(dst2.at[pl.ds(gw, PWG)], dstbuf.at[lax.rem(w, 3)], isem)

    def wait_idx(w):
        eb = gbase * GRP + w * PW
        gw = gbase + w * PWG
        pltpu.make_async_copy(src1.at[pl.ds(eb, PW)],
                              srcbuf.at[lax.rem(w, 2)], isem).wait()
        pltpu.make_async_copy(dst2.at[pl.ds(gw, PWG)],
                              dstbuf.at[lax.rem(w, 3)], isem).wait()

    def fire_gathers(c, w):
        b = lax.rem(w, 2)
        ytab = yflat.at[pl.ds(pl.multiple_of(c * NP1, 8), NP1)]
        pltpu.async_copy(ytab.at[srcbuf.at[b]], rows.at[b], gsem)

    def drain_gathers(c, w):
        b = lax.rem(w, 2)
        ytab = yflat.at[pl.ds(pl.multiple_of(c * NP1, 8), NP1)]
        pltpu.make_async_copy(ytab.at[srcbuf.at[b]], rows.at[b], gsem).wait()

    def fire_scatters(w):
        b, b3 = lax.rem(w, 2), lax.rem(w, 3)
        for j in range(PWG):
            pltpu.async_copy(rows.at[b, pl.ds(j * GRP, GRP)],
                             acc.at[dstbuf.at[b3, j]], ssem, add=True)

    def drain_scatters(w):
        b, b3 = lax.rem(w, 2), lax.rem(w, 3)
        for j in range(PWG):
            pltpu.make_async_copy(rows.at[b, pl.ds(j * GRP, GRP)],
                                  acc.at[dstbuf.at[b3, j]], ssem).wait()

    def chunk(c, _):
        _zero_loop(zbuf, 512, 16)
        for off, sz in SLICESZ:
            pltpu.sync_copy(zbuf.at[pl.ds(0, sz)], acc.at[pl.ds(zbase + off, sz)])
        plsc.subcore_barrier()

        issue_idx(0)
        wait_idx(0)
        fire_gathers(c, 0)
        issue_idx(1)

        def window(w, _):
            @pl.when(w > 0)
            def _ds():
                drain_scatters(w - 1)

            drain_gathers(c, w)
            fire_scatters(w)

            @pl.when(w + 1 < PWIN)
            def _next():
                wait_idx(w + 1)
                fire_gathers(c, w + 1)

                @pl.when(w + 2 < PWIN)
                def _ii():
                    issue_idx(w + 2)
            return _

        lax.fori_loop(0, PWIN, window, 0)
        drain_scatters(PWIN - 1)
        plsc.subcore_barrier()
        for off, sz in SLICESZ:
            pltpu.sync_copy(acc.at[pl.ds(zbase + off, sz)],
                            zbuf.at[pl.ds(0, sz)])
            pltpu.sync_copy(zbuf.at[pl.ds(0, sz)],
                            out.at[core, pl.ds(zbase + off, sz),
                                   pl.ds(c * 16, 16)])
        plsc.subcore_barrier()
        return _

    lax.fori_loop(0, nchunk, chunk, 0)


def _make_prop_kernel(nchunk):
    mesh = plsc.VectorSubcoreMesh(core_axis_name="c", subcore_axis_name="s")
    return pl.kernel(
        functools.partial(_prop_body, nchunk),
        out_type=jax.ShapeDtypeStruct((NSC, NACC, 16 * nchunk), jnp.float32),
        name="prop%d" % nchunk,
        mesh=mesh,
        compiler_params=pltpu.CompilerParams(use_tc_tiling_on_sc=False),
        scratch_types=[
            pltpu.VMEM_SHARED((NACC, 16), jnp.float32),
            pltpu.VMEM((2, PW), jnp.int32),
            pltpu.VMEM((3, PWG, GRP), jnp.int32),
            pltpu.VMEM((2, PW, 16), jnp.float32),
            pltpu.VMEM((512, 16), jnp.float32),
            pltpu.SemaphoreType.DMA,
            pltpu.SemaphoreType.DMA,
            pltpu.SemaphoreType.DMA,
        ],
    )


PR = 8192
LR = 4096


def _prologue_body(deg0, deg1, x, dinv, y1):
    deg = deg0[...] + deg1[...] + 1.0
    dv = lax.rsqrt(deg)
    dinv[...] = dv
    y = x[...] * dv
    y1[...] = jnp.concatenate(
        [y, jnp.zeros((y.shape[0], 16 - y.shape[1]), jnp.float32)], axis=1)


def _tc_prologue(deg0, deg1, x):
    grid = (pl.cdiv(NP1, PR),)
    return pl.pallas_call(
        _prologue_body,
        grid=grid,
        in_specs=[
            pl.BlockSpec((PR, 1), lambda i: (i, 0)),
            pl.BlockSpec((PR, 1), lambda i: (i, 0)),
            pl.BlockSpec((PR, 5), lambda i: (i, 0)),
        ],
        out_specs=[
            pl.BlockSpec((PR, 1), lambda i: (i, 0)),
            pl.BlockSpec((PR, 16), lambda i: (i, 0)),
        ],
        out_shape=[
            jax.ShapeDtypeStruct((NP1, 1), jnp.float32),
            jax.ShapeDtypeStruct((NP1, 16), jnp.float32),
        ],
    )(deg0, deg1, x)


def _layer_body(scale_out, accp, y, dinv, w, b, out):
    s = (accp[0] + accp[1] + y[...]) * dinv[...]
    z = jnp.dot(s, w[...], preferred_element_type=jnp.float32) + b[...]
    h = jnp.maximum(z, 0.0)
    if scale_out:
        h = h * dinv[...]
    out[...] = h


def _tc_layer(accp, y, dinv, w, b, scale_out):
    k = y.shape[1]
    dout = w.shape[1]
    grid = (pl.cdiv(NP1, LR),)
    return pl.pallas_call(
        functools.partial(_layer_body, scale_out),
        grid=grid,
        in_specs=[
            pl.BlockSpec((NSC, LR, k), lambda i: (0, i, 0)),
            pl.BlockSpec((LR, k), lambda i: (i, 0)),
            pl.BlockSpec((LR, 1), lambda i: (i, 0)),
            pl.BlockSpec((k, dout), lambda i: (0, 0)),
            pl.BlockSpec((1, dout), lambda i: (0, 0)),
        ],
        out_specs=pl.BlockSpec((LR, dout), lambda i: (i, 0)),
        out_shape=jax.ShapeDtypeStruct((NP1, dout), jnp.float32),
    )(accp, y, dinv, w, b)


def _pool_body(nblk, h, batch, wfc, bfc, out, sums, counts):
    i = pl.program_id(0)

    @pl.when(i == 0)
    def _init():
        sums[...] = jnp.zeros_like(sums)
        counts[...] = jnp.zeros_like(counts)

    bb = batch[...]
    gid = lax.broadcasted_iota(jnp.int32, (LR, G), 1)
    rowid = i * LR + lax.broadcasted_iota(jnp.int32, (LR, G), 0)
    onehot = jnp.where((bb == gid) & (rowid < N), 1.0, 0.0)
    valid = (i * LR + lax.broadcasted_iota(jnp.int32, (LR, 1), 0)) < N
    hv = jnp.where(valid, h[...], 0.0)
    dn = (((0,), (0,)), ((), ()))
    sums[...] += lax.dot_general(onehot, hv, dn,
                                 preferred_element_type=jnp.float32)
    counts[...] += lax.dot_general(onehot, jnp.ones((LR, 1), jnp.float32), dn,
                                   preferred_element_type=jnp.float32)

    @pl.when(i == nblk - 1)
    def _final():
        pooled = sums[...] / jnp.maximum(counts[...], 1.0)
        z = jnp.dot(pooled, wfc[...], preferred_element_type=jnp.float32) + bfc[...]
        m = jnp.max(z, axis=1, keepdims=True)
        e = jnp.exp(z - m)
        lse = jnp.log(jnp.sum(e, axis=1, keepdims=True)) + m
        out[...] = z - lse


def _tc_poolfc(h, batch2, wfc, bfc):
    nblk = pl.cdiv(NP1, LR)
    return pl.pallas_call(
        functools.partial(_pool_body, nblk),
        grid=(nblk,),
        in_specs=[
            pl.BlockSpec((LR, 256), lambda i: (i, 0)),
            pl.BlockSpec((LR, 1), lambda i: (i, 0)),
            pl.BlockSpec((256, NCLS), lambda i: (0, 0)),
            pl.BlockSpec((1, NCLS), lambda i: (0, 0)),
        ],
        out_specs=pl.BlockSpec((G, NCLS), lambda i: (0, 0)),
        out_shape=jax.ShapeDtypeStruct((G, NCLS), jnp.float32),
        scratch_shapes=[
            pltpu.VMEM((G, 256), jnp.float32),
            pltpu.VMEM((G, 1), jnp.float32),
        ],
    )(h, batch2, wfc, bfc)


def kernel(x, edge_index, batch, W1, b1, W2, b2, W3, b3, W4, b4, Wfc, bfc):
    src = edge_index[0]
    dst = edge_index[1]
    npad = E_PAD - E
    padi = jnp.arange(npad, dtype=jnp.int32)
    srcp = jnp.concatenate([src, N + (padi % 8)])
    dstp = jnp.concatenate([dst, N + (padi % 16)]).reshape(EG_ROWS, GRP)

    deg0, deg1 = _make_deg_kernel()(dstp)
    dinv, y = _tc_prologue(deg0.reshape(NDEG, 1), deg1.reshape(NDEG, 1), x)

    weights = [(jnp.pad(W1, ((0, 11), (0, 0))), b1), (W2, b2), (W3, b3), (W4, b4)]
    for li, (w, b) in enumerate(weights):
        k = w.shape[0]
        nchunk = k // 16
        if nchunk == 1:
            yflat = y
        else:
            yflat = y.reshape(NP1, nchunk, 16).transpose(1, 0, 2)
            yflat = yflat.reshape(NP1 * nchunk, 16)
        accp = _make_prop_kernel(nchunk)(yflat, srcp, dstp)
        y = _tc_layer(accp, y, dinv, w, b.reshape(1, -1), scale_out=(li < 3))

    return _tc_poolfc(y, batch.reshape(N, 1), Wfc, bfc.reshape(1, NCLS))


# speedup vs baseline: 1.1301x; 1.1301x over previous
import functools

import jax
import jax.numpy as jnp
from jax import lax
from jax.experimental import pallas as pl
from jax.experimental.pallas import tpu as pltpu
from jax.experimental.pallas import tpu_sc as plsc

N = 100000
E = 6400000
G = 64
NCLS = 10

NSC = 2
NTILE = 16
GRP = 128
WG = 16
W = GRP * WG
E_PAD = 6422528
WINDOWS = E_PAD // (NSC * NTILE * W)
EG_ROWS = E_PAD // GRP
PWG = 4
PW = GRP * PWG
PWIN = E_PAD // (NSC * NTILE * PW)

NP1 = 100008
NACC = 100096
ACC_T = 6256
NDEG = 100096
DEG_T = 6256
SLICESZ = tuple((k * 512, 512) for k in range(12)) + ((6144, 112),)

ZROWS = 1024


def _zero_loop(zbuf, nrows, width):
    zv = jnp.zeros((16,), jnp.float32)

    def body(i, _):
        r = i // (width // 16)
        k = i % (width // 16)
        zbuf[r, pl.ds(k * 16, 16)] = zv
        return _

    lax.fori_loop(0, nrows * (width // 16), body, 0)


def _deg_body(dst2, out0, out1, acc, dstbuf, ones_v, zbuf, ssem):
    core = lax.axis_index("c")
    tile = lax.axis_index("s")

    zv = jnp.zeros((16,), jnp.float32)
    ov = jnp.ones((16,), jnp.float32)

    def stage(i, _):
        zbuf[pl.ds(i * 16, 16)] = zv
        return _

    lax.fori_loop(0, DEG_T // 16, stage, 0)

    def ones_body(i, _):
        ones_v[pl.ds(i * 16, 16)] = ov
        return _

    lax.fori_loop(0, GRP // 16, ones_body, 0)

    base = tile * DEG_T
    pltpu.sync_copy(zbuf.at[pl.ds(0, DEG_T)], acc.at[pl.ds(base, DEG_T)])
    plsc.subcore_barrier()

    gbase = (core * NTILE + tile) * (WINDOWS * WG)

    def window(w, _):
        gw = gbase + w * WG
        pltpu.sync_copy(dst2.at[pl.ds(gw, WG)], dstbuf)

        def fire(j, _):
            pltpu.async_copy(ones_v, acc.at[dstbuf.at[j]], ssem, add=True)
            return _

        lax.fori_loop(0, WG, fire, 0)

        def drain(j, _):
            pltpu.make_async_copy(ones_v, acc.at[dstbuf.at[j]], ssem).wait()
            return _

        lax.fori_loop(0, WG, drain, 0)
        return _

    lax.fori_loop(0, WINDOWS, window, 0)
    plsc.subcore_barrier()
    pltpu.sync_copy(acc.at[pl.ds(base, DEG_T)], zbuf)
    for k, o in enumerate((out0, out1)):
        @pl.when(core == k)
        def _copy(o=o):
            pltpu.sync_copy(zbuf, o.at[pl.ds(base, DEG_T)])


def _make_deg_kernel():
    mesh = plsc.VectorSubcoreMesh(core_axis_name="c", subcore_axis_name="s")
    return pl.kernel(
        _deg_body,
        out_type=[jax.ShapeDtypeStruct((NDEG,), jnp.float32),
                  jax.ShapeDtypeStruct((NDEG,), jnp.float32)],
        mesh=mesh,
        scratch_types=[
            pltpu.VMEM_SHARED((NDEG,), jnp.float32),
            pltpu.VMEM((WG, GRP), jnp.int32),
            pltpu.VMEM((GRP,), jnp.float32),
            pltpu.VMEM((DEG_T,), jnp.float32),
            pltpu.SemaphoreType.DMA,
        ],
    )


def _prop_body(nchunk, yflat, src1, dst2, out, acc, srcbuf, dstbuf, rows,
               zbuf, isem, gsem, ssem):
    core = lax.axis_index("c")
    tile = lax.axis_index("s")
    gbase = (core * NTILE + tile) * (PWIN * PWG)
    zbase = tile * ACC_T

    def issue_idx(w):
        eb = gbase * GRP + w * PW
        gw = gbase + w * PWG
        pltpu.async_copy(src1.at[pl.ds(eb, PW)], srcbuf.at[lax.rem(w, 2)], isem)
        pltpu.async_copy(dst2.at[pl.ds(gw, PWG)], dstbuf.at[lax.rem(w, 3)], isem)

    def wait_idx(w):
        eb = gbase * GRP + w * PW
        gw = gbase + w * PWG
        pltpu.make_async_copy(src1.at[pl.ds(eb, PW)],
                              srcbuf.at[lax.rem(w, 2)], isem).wait()
        pltpu.make_async_copy(dst2.at[pl.ds(gw, PWG)],
                              dstbuf.at[lax.rem(w, 3)], isem).wait()

    def compute_gidx(c, w):
        if nchunk > 1:
            b = lax.rem(w, 2)

            def gidx(i, _):
                v = srcbuf[b, pl.ds(i * 16, 16)]
                srcbuf[b, pl.ds(i * 16, 16)] = v * nchunk + c
                return _

            lax.fori_loop(0, PW // 16, gidx, 0)

    def fire_gathers(c, w):
        b = lax.rem(w, 2)
        pltpu.async_copy(yflat.at[srcbuf.at[b]], rows.at[b], gsem)

    def drain_gathers(c, w):
        b = lax.rem(w, 2)
        pltpu.make_async_copy(yflat.at[srcbuf.at[b]], rows.at[b], gsem).wait()

    def fire_scatters(w):
        b, b3 = lax.rem(w, 2), lax.rem(w, 3)
        for j in range(PWG):
            pltpu.async_copy(rows.at[b, pl.ds(j * GRP, GRP)],
                             acc.at[dstbuf.at[b3, j]], ssem, add=True)

    def drain_scatters(w):
        b, b3 = lax.rem(w, 2), lax.rem(w, 3)
        for j in range(PWG):
            pltpu.make_async_copy(rows.at[b, pl.ds(j * GRP, GRP)],
                                  acc.at[dstbuf.at[b3, j]], ssem).wait()

    def chunk(c, _):
        _zero_loop(zbuf, 512, 16)
        for off, sz in SLICESZ:
            pltpu.sync_copy(zbuf.at[pl.ds(0, sz)], acc.at[pl.ds(zbase + off, sz)])
        plsc.subcore_barrier()

        issue_idx(0)
        wait_idx(0)
        compute_gidx(c, 0)
        fire_gathers(c, 0)
        issue_idx(1)

        def window(w, _):
            @pl.when(w > 0)
            def _ds():
                drain_scatters(w - 1)

            drain_gathers(c, w)
            fire_scatters(w)

            @pl.when(w + 1 < PWIN)
            def _next():
                wait_idx(w + 1)
                compute_gidx(c, w + 1)
                fire_gathers(c, w + 1)

                @pl.when(w + 2 < PWIN)
                def _ii():
                    issue_idx(w + 2)
            return _

        lax.fori_loop(0, PWIN, window, 0)
        drain_scatters(PWIN - 1)
        plsc.subcore_barrier()
        for off, sz in SLICESZ:
            pltpu.sync_copy(acc.at[pl.ds(zbase + off, sz)],
                            zbuf.at[pl.ds(0, sz)])
            pltpu.sync_copy(zbuf.at[pl.ds(0, sz)],
                            out.at[core, pl.ds(zbase + off, sz),
                                   pl.ds(c * 16, 16)])
        plsc.subcore_barrier()
        return _

    lax.fori_loop(0, nchunk, chunk, 0)


def _make_prop_kernel(nchunk):
    mesh = plsc.VectorSubcoreMesh(core_axis_name="c", subcore_axis_name="s")
    return pl.kernel(
        functools.partial(_prop_body, nchunk),
        out_type=jax.ShapeDtypeStruct((NSC, NACC, 16 * nchunk), jnp.float32),
        name="prop%d" % nchunk,
        mesh=mesh,
        compiler_params=pltpu.CompilerParams(use_tc_tiling_on_sc=False),
        scratch_types=[
            pltpu.VMEM_SHARED((NACC, 16), jnp.float32),
            pltpu.VMEM((2, PW), jnp.int32),
            pltpu.VMEM((3, PWG, GRP), jnp.int32),
            pltpu.VMEM((2, PW, 16), jnp.float32),
            pltpu.VMEM((512, 16), jnp.float32),
            pltpu.SemaphoreType.DMA,
            pltpu.SemaphoreType.DMA,
            pltpu.SemaphoreType.DMA,
        ],
    )


PR = 8192
LR = 4096


def _prologue_body(deg0, deg1, x, dinv, y1):
    deg = deg0[...] + deg1[...] + 1.0
    dv = lax.rsqrt(deg)
    dinv[...] = dv
    y = x[...] * dv
    y1[...] = jnp.concatenate(
        [y, jnp.zeros((y.shape[0], 16 - y.shape[1]), jnp.float32)], axis=1)


def _tc_prologue(deg0, deg1, x):
    grid = (pl.cdiv(NP1, PR),)
    return pl.pallas_call(
        _prologue_body,
        grid=grid,
        in_specs=[
            pl.BlockSpec((PR, 1), lambda i: (i, 0)),
            pl.BlockSpec((PR, 1), lambda i: (i, 0)),
            pl.BlockSpec((PR, 5), lambda i: (i, 0)),
        ],
        out_specs=[
            pl.BlockSpec((PR, 1), lambda i: (i, 0)),
            pl.BlockSpec((PR, 16), lambda i: (i, 0)),
        ],
        out_shape=[
            jax.ShapeDtypeStruct((NP1, 1), jnp.float32),
            jax.ShapeDtypeStruct((NP1, 16), jnp.float32),
        ],
    )(deg0, deg1, x)


def _layer_body(scale_out, accp, y, dinv, w, b, out):
    s = (accp[0] + accp[1] + y[...]) * dinv[...]
    z = jnp.dot(s, w[...], preferred_element_type=jnp.float32) + b[...]
    h = jnp.maximum(z, 0.0)
    if scale_out:
        h = h * dinv[...]
    out[...] = h


def _tc_layer(accp, y, dinv, w, b, scale_out):
    k = y.shape[1]
    dout = w.shape[1]
    grid = (pl.cdiv(NP1, LR),)
    return pl.pallas_call(
        functools.partial(_layer_body, scale_out),
        grid=grid,
        in_specs=[
            pl.BlockSpec((NSC, LR, k), lambda i: (0, i, 0)),
            pl.BlockSpec((LR, k), lambda i: (i, 0)),
            pl.BlockSpec((LR, 1), lambda i: (i, 0)),
            pl.BlockSpec((k, dout), lambda i: (0, 0)),
            pl.BlockSpec((1, dout), lambda i: (0, 0)),
        ],
        out_specs=pl.BlockSpec((LR, dout), lambda i: (i, 0)),
        out_shape=jax.ShapeDtypeStruct((NP1, dout), jnp.float32),
    )(accp, y, dinv, w, b)


def _pool_body(nblk, h, batch, wfc, bfc, out, sums, counts):
    i = pl.program_id(0)

    @pl.when(i == 0)
    def _init():
        sums[...] = jnp.zeros_like(sums)
        counts[...] = jnp.zeros_like(counts)

    bb = batch[...]
    gid = lax.broadcasted_iota(jnp.int32, (LR, G), 1)
    rowid = i * LR + lax.broadcasted_iota(jnp.int32, (LR, G), 0)
    onehot = jnp.where((bb == gid) & (rowid < N), 1.0, 0.0)
    valid = (i * LR + lax.broadcasted_iota(jnp.int32, (LR, 1), 0)) < N
    hv = jnp.where(valid, h[...], 0.0)
    dn = (((0,), (0,)), ((), ()))
    sums[...] += lax.dot_general(onehot, hv, dn,
                                 preferred_element_type=jnp.float32)
    counts[...] += lax.dot_general(onehot, jnp.ones((LR, 1), jnp.float32), dn,
                                   preferred_element_type=jnp.float32)

    @pl.when(i == nblk - 1)
    def _final():
        pooled = sums[...] / jnp.maximum(counts[...], 1.0)
        z = jnp.dot(pooled, wfc[...], preferred_element_type=jnp.float32) + bfc[...]
        m = jnp.max(z, axis=1, keepdims=True)
        e = jnp.exp(z - m)
        lse = jnp.log(jnp.sum(e, axis=1, keepdims=True)) + m
        out[...] = z - lse


def _tc_poolfc(h, batch2, wfc, bfc):
    nblk = pl.cdiv(NP1, LR)
    return pl.pallas_call(
        functools.partial(_pool_body, nblk),
        grid=(nblk,),
        in_specs=[
            pl.BlockSpec((LR, 256), lambda i: (i, 0)),
            pl.BlockSpec((LR, 1), lambda i: (i, 0)),
            pl.BlockSpec((256, NCLS), lambda i: (0, 0)),
            pl.BlockSpec((1, NCLS), lambda i: (0, 0)),
        ],
        out_specs=pl.BlockSpec((G, NCLS), lambda i: (0, 0)),
        out_shape=jax.ShapeDtypeStruct((G, NCLS), jnp.float32),
        scratch_shapes=[
            pltpu.VMEM((G, 256), jnp.float32),
            pltpu.VMEM((G, 1), jnp.float32),
        ],
    )(h, batch2, wfc, bfc)


def kernel(x, edge_index, batch, W1, b1, W2, b2, W3, b3, W4, b4, Wfc, bfc):
    src = edge_index[0]
    dst = edge_index[1]
    npad = E_PAD - E
    padi = jnp.arange(npad, dtype=jnp.int32)
    srcp = jnp.concatenate([src, N + (padi % 8)])
    dstp = jnp.concatenate([dst, N + (padi % 16)]).reshape(EG_ROWS, GRP)

    deg0, deg1 = _make_deg_kernel()(dstp)
    dinv, y = _tc_prologue(deg0.reshape(NDEG, 1), deg1.reshape(NDEG, 1), x)

    weights = [(jnp.pad(W1, ((0, 11), (0, 0))), b1), (W2, b2), (W3, b3), (W4, b4)]
    for li, (w, b) in enumerate(weights):
        k = w.shape[0]
        nchunk = k // 16
        yflat = y.reshape(NP1 * nchunk, 16)
        accp = _make_prop_kernel(nchunk)(yflat, srcp, dstp)
        y = _tc_layer(accp, y, dinv, w, b.reshape(1, -1), scale_out=(li < 3))

    return _tc_poolfc(y, batch.reshape(N, 1), Wfc, bfc.reshape(1, NCLS))


# speedup vs baseline: 1.1381x; 1.0070x over previous
import functools

import jax
import jax.numpy as jnp
from jax import lax
from jax.experimental import pallas as pl
from jax.experimental.pallas import tpu as pltpu
from jax.experimental.pallas import tpu_sc as plsc

N = 100000
E = 6400000
G = 64
NCLS = 10

NSC = 2
NTILE = 16
GRP = 128
WG = 16
W = GRP * WG
E_PAD = 6488064
NPAD = E_PAD - E
WINDOWS = E_PAD // (NSC * NTILE * W)
EG_ROWS = E_PAD // GRP
PWG = 6
PW = GRP * PWG
PWIN = E_PAD // (NSC * NTILE * PW)

NP1 = 100008
NACC = 100096
ACC_T = 6256
NDEG = 100096
DEG_T = 6256
SLICESZ = tuple((k * 144, 144) for k in range(43)) + ((6192, 64),)

ZROWS = 1024


def _zero_loop(zbuf, nrows, width):
    zv = jnp.zeros((16,), jnp.float32)

    def body(i, _):
        r = i // (width // 16)
        k = i % (width // 16)
        zbuf[r, pl.ds(k * 16, 16)] = zv
        return _

    lax.fori_loop(0, nrows * (width // 16), body, 0)


def _deg_body(dst2, out0, out1, acc, dstbuf, ones_v, zbuf, ssem):
    core = lax.axis_index("c")
    tile = lax.axis_index("s")

    zv = jnp.zeros((16,), jnp.float32)
    ov = jnp.ones((16,), jnp.float32)

    def stage(i, _):
        zbuf[pl.ds(i * 16, 16)] = zv
        return _

    lax.fori_loop(0, DEG_T // 16, stage, 0)

    def ones_body(i, _):
        ones_v[pl.ds(i * 16, 16)] = ov
        return _

    lax.fori_loop(0, GRP // 16, ones_body, 0)

    base = tile * DEG_T
    pltpu.sync_copy(zbuf.at[pl.ds(0, DEG_T)], acc.at[pl.ds(base, DEG_T)])
    plsc.subcore_barrier()

    gbase = (core * NTILE + tile) * (WINDOWS * WG)

    def window(w, _):
        gw = gbase + w * WG
        pltpu.sync_copy(dst2.at[pl.ds(gw, WG)], dstbuf)

        def fire(j, _):
            pltpu.async_copy(ones_v, acc.at[dstbuf.at[j]], ssem, add=True)
            return _

        lax.fori_loop(0, WG, fire, 0)

        def drain(j, _):
            pltpu.make_async_copy(ones_v, acc.at[dstbuf.at[j]], ssem).wait()
            return _

        lax.fori_loop(0, WG, drain, 0)
        return _

    lax.fori_loop(0, WINDOWS, window, 0)
    plsc.subcore_barrier()
    pltpu.sync_copy(acc.at[pl.ds(base, DEG_T)], zbuf)
    for k, o in enumerate((out0, out1)):
        @pl.when(core == k)
        def _copy(o=o):
            pltpu.sync_copy(zbuf, o.at[pl.ds(base, DEG_T)])


def _make_deg_kernel():
    mesh = plsc.VectorSubcoreMesh(core_axis_name="c", subcore_axis_name="s")
    return pl.kernel(
        _deg_body,
        out_type=[jax.ShapeDtypeStruct((NDEG,), jnp.float32),
                  jax.ShapeDtypeStruct((NDEG,), jnp.float32)],
        mesh=mesh,
        scratch_types=[
            pltpu.VMEM_SHARED((NDEG,), jnp.float32),
            pltpu.VMEM((WG, GRP), jnp.int32),
            pltpu.VMEM((GRP,), jnp.float32),
            pltpu.VMEM((DEG_T,), jnp.float32),
            pltpu.SemaphoreType.DMA,
        ],
    )


def _prop_body(nchunk, yflat, src1, dst2, out, acc, srcbuf, dstbuf, rows,
               zbuf, isem, gsem, ssem):
    core = lax.axis_index("c")
    tile = lax.axis_index("s")
    gbase = (core * NTILE + tile) * (PWIN * PWG)
    zbase = tile * ACC_T

    def issue_idx(w):
        eb = gbase * GRP + w * PW
        gw = gbase + w * PWG
        pltpu.async_copy(src1.at[pl.ds(eb, PW)], srcbuf.at[lax.rem(w, 2)], isem)
        pltpu.async_copy(dst2.at[pl.ds(gw, PWG)], dstbuf.at[lax.rem(w, 3)], isem)

    def wait_idx(w):
        eb = gbase * GRP + w * PW
        gw = gbase + w * PWG
        pltpu.make_async_copy(src1.at[pl.ds(eb, PW)],
                              srcbuf.at[lax.rem(w, 2)], isem).wait()
        pltpu.make_async_copy(dst2.at[pl.ds(gw, PWG)],
                              dstbuf.at[lax.rem(w, 3)], isem).wait()

    def compute_gidx(c, w):
        if nchunk > 1:
            b = lax.rem(w, 2)

            def gidx(i, _):
                v = srcbuf[b, pl.ds(i * 16, 16)]
                srcbuf[b, pl.ds(i * 16, 16)] = v * nchunk + c
                return _

            lax.fori_loop(0, PW // 16, gidx, 0)

    def fire_gathers(c, w):
        b = lax.rem(w, 2)
        pltpu.async_copy(yflat.at[srcbuf.at[b]], rows.at[b], gsem)

    def drain_gathers(c, w):
        b = lax.rem(w, 2)
        pltpu.make_async_copy(yflat.at[srcbuf.at[b]], rows.at[b], gsem).wait()

    def fire_scatters(w):
        b, b3 = lax.rem(w, 2), lax.rem(w, 3)
        for j in range(PWG):
            pltpu.async_copy(rows.at[b, pl.ds(j * GRP, GRP)],
                             acc.at[dstbuf.at[b3, j]], ssem, add=True)

    def drain_scatters(w):
        b, b3 = lax.rem(w, 2), lax.rem(w, 3)
        for j in range(PWG):
            pltpu.make_async_copy(rows.at[b, pl.ds(j * GRP, GRP)],
                                  acc.at[dstbuf.at[b3, j]], ssem).wait()

    def chunk(c, _):
        _zero_loop(zbuf, 144, 16)
        for off, sz in SLICESZ:
            pltpu.sync_copy(zbuf.at[pl.ds(0, sz)], acc.at[pl.ds(zbase + off, sz)])
        plsc.subcore_barrier()

        issue_idx(0)
        wait_idx(0)
        compute_gidx(c, 0)
        fire_gathers(c, 0)
        issue_idx(1)

        def window(w, _):
            @pl.when(w > 0)
            def _ds():
                drain_scatters(w - 1)

            drain_gathers(c, w)
            fire_scatters(w)

            @pl.when(w + 1 < PWIN)
            def _next():
                wait_idx(w + 1)
                compute_gidx(c, w + 1)
                fire_gathers(c, w + 1)

                @pl.when(w + 2 < PWIN)
                def _ii():
                    issue_idx(w + 2)
            return _

        lax.fori_loop(0, PWIN, window, 0)
        drain_scatters(PWIN - 1)
        plsc.subcore_barrier()
        for off, sz in SLICESZ:
            pltpu.sync_copy(acc.at[pl.ds(zbase + off, sz)],
                            zbuf.at[pl.ds(0, sz)])
            pltpu.sync_copy(zbuf.at[pl.ds(0, sz)],
                            out.at[core, pl.ds(zbase + off, sz),
                                   pl.ds(c * 16, 16)])
        plsc.subcore_barrier()
        return _

    lax.fori_loop(0, nchunk, chunk, 0)


def _make_prop_kernel(nchunk):
    mesh = plsc.VectorSubcoreMesh(core_axis_name="c", subcore_axis_name="s")
    return pl.kernel(
        functools.partial(_prop_body, nchunk),
        out_type=jax.ShapeDtypeStruct((NSC, NACC, 16 * nchunk), jnp.float32),
        name="prop%d" % nchunk,
        mesh=mesh,
        compiler_params=pltpu.CompilerParams(use_tc_tiling_on_sc=False),
        scratch_types=[
            pltpu.VMEM_SHARED((NACC, 16), jnp.float32),
            pltpu.VMEM((2, PW), jnp.int32),
            pltpu.VMEM((3, PWG, GRP), jnp.int32),
            pltpu.VMEM((2, PW, 16), jnp.float32),
            pltpu.VMEM((144, 16), jnp.float32),
            pltpu.SemaphoreType.DMA,
            pltpu.SemaphoreType.DMA,
            pltpu.SemaphoreType.DMA,
        ],
    )


PR = 8192
LR = 4096


def _prologue_body(deg0, deg1, x, dinv, y1):
    i = pl.program_id(0)
    rowid = i * PR + lax.broadcasted_iota(jnp.int32, (PR, 1), 0)
    padsub = jnp.where(rowid < NPAD, 1.0, 0.0)
    deg = deg0[...] + deg1[...] + 1.0 - padsub
    dv = lax.rsqrt(deg)
    dinv[...] = dv
    y = jnp.where(rowid < N, x[...] * dv, 0.0)
    y1[...] = jnp.concatenate(
        [y, jnp.zeros((y.shape[0], 16 - y.shape[1]), jnp.float32)], axis=1)


def _tc_prologue(deg0, deg1, x):
    grid = (pl.cdiv(NP1, PR),)
    return pl.pallas_call(
        _prologue_body,
        grid=grid,
        in_specs=[
            pl.BlockSpec((PR, 1), lambda i: (i, 0)),
            pl.BlockSpec((PR, 1), lambda i: (i, 0)),
            pl.BlockSpec((PR, 5), lambda i: (i, 0)),
        ],
        out_specs=[
            pl.BlockSpec((PR, 1), lambda i: (i, 0)),
            pl.BlockSpec((PR, 16), lambda i: (i, 0)),
        ],
        out_shape=[
            jax.ShapeDtypeStruct((NP1, 1), jnp.float32),
            jax.ShapeDtypeStruct((NP1, 16), jnp.float32),
        ],
    )(deg0, deg1, x)


def _layer_body(scale_out, accp, y, dinv, w, b, out):
    i = pl.program_id(0)
    rowid = i * LR + lax.broadcasted_iota(jnp.int32, (LR, 1), 0)
    s = (accp[0] + accp[1] + y[...]) * dinv[...]
    z = jnp.dot(s, w[...], preferred_element_type=jnp.float32) + b[...]
    h = jnp.maximum(z, 0.0)
    if scale_out:
        h = h * dinv[...]
    out[...] = jnp.where(rowid < N, h, 0.0)


def _tc_layer(accp, y, dinv, w, b, scale_out):
    k = y.shape[1]
    dout = w.shape[1]
    grid = (pl.cdiv(NP1, LR),)
    return pl.pallas_call(
        functools.partial(_layer_body, scale_out),
        grid=grid,
        in_specs=[
            pl.BlockSpec((NSC, LR, k), lambda i: (0, i, 0)),
            pl.BlockSpec((LR, k), lambda i: (i, 0)),
            pl.BlockSpec((LR, 1), lambda i: (i, 0)),
            pl.BlockSpec((k, dout), lambda i: (0, 0)),
            pl.BlockSpec((1, dout), lambda i: (0, 0)),
        ],
        out_specs=pl.BlockSpec((LR, dout), lambda i: (i, 0)),
        out_shape=jax.ShapeDtypeStruct((NP1, dout), jnp.float32),
    )(accp, y, dinv, w, b)


def _pool_body(nblk, h, batch, wfc, bfc, out, sums, counts):
    i = pl.program_id(0)

    @pl.when(i == 0)
    def _init():
        sums[...] = jnp.zeros_like(sums)
        counts[...] = jnp.zeros_like(counts)

    bb = batch[...]
    gid = lax.broadcasted_iota(jnp.int32, (LR, G), 1)
    rowid = i * LR + lax.broadcasted_iota(jnp.int32, (LR, G), 0)
    onehot = jnp.where((bb == gid) & (rowid < N), 1.0, 0.0)
    valid = (i * LR + lax.broadcasted_iota(jnp.int32, (LR, 1), 0)) < N
    hv = jnp.where(valid, h[...], 0.0)
    dn = (((0,), (0,)), ((), ()))
    sums[...] += lax.dot_general(onehot, hv, dn,
                                 preferred_element_type=jnp.float32)
    counts[...] += lax.dot_general(onehot, jnp.ones((LR, 1), jnp.float32), dn,
                                   preferred_element_type=jnp.float32)

    @pl.when(i == nblk - 1)
    def _final():
        pooled = sums[...] / jnp.maximum(counts[...], 1.0)
        z = jnp.dot(pooled, wfc[...], preferred_element_type=jnp.float32) + bfc[...]
        m = jnp.max(z, axis=1, keepdims=True)
        e = jnp.exp(z - m)
        lse = jnp.log(jnp.sum(e, axis=1, keepdims=True)) + m
        out[...] = z - lse


def _tc_poolfc(h, batch2, wfc, bfc):
    nblk = pl.cdiv(NP1, LR)
    return pl.pallas_call(
        functools.partial(_pool_body, nblk),
        grid=(nblk,),
        in_specs=[
            pl.BlockSpec((LR, 256), lambda i: (i, 0)),
            pl.BlockSpec((LR, 1), lambda i: (i, 0)),
            pl.BlockSpec((256, NCLS), lambda i: (0, 0)),
            pl.BlockSpec((1, NCLS), lambda i: (0, 0)),
        ],
        out_specs=pl.BlockSpec((G, NCLS), lambda i: (0, 0)),
        out_shape=jax.ShapeDtypeStruct((G, NCLS), jnp.float32),
        scratch_shapes=[
            pltpu.VMEM((G, 256), jnp.float32),
            pltpu.VMEM((G, 1), jnp.float32),
        ],
    )(h, batch2, wfc, bfc)


def kernel(x, edge_index, batch, W1, b1, W2, b2, W3, b3, W4, b4, Wfc, bfc):
    src = edge_index[0]
    dst = edge_index[1]
    padi = jnp.arange(NPAD, dtype=jnp.int32)
    srcp = jnp.concatenate([src, N + (padi % 8)])
    dstp = jnp.concatenate([dst, padi]).reshape(EG_ROWS, GRP)

    deg0, deg1 = _make_deg_kernel()(dstp)
    dinv, y = _tc_prologue(deg0.reshape(NDEG, 1), deg1.reshape(NDEG, 1), x)

    weights = [(jnp.pad(W1, ((0, 11), (0, 0))), b1), (W2, b2), (W3, b3), (W4, b4)]
    for li, (w, b) in enumerate(weights):
        k = w.shape[0]
        nchunk = k // 16
        yflat = y.reshape(NP1 * nchunk, 16)
        accp = _make_prop_kernel(nchunk)(yflat, srcp, dstp)
        y = _tc_layer(accp, y, dinv, w, b.reshape(1, -1), scale_out=(li < 3))

    return _tc_poolfc(y, batch.reshape(N, 1), Wfc, bfc.reshape(1, NCLS))


# speedup vs baseline: 1.2134x; 1.0662x over previous
import functools

import jax
import jax.numpy as jnp
from jax import lax
from jax.experimental import pallas as pl
from jax.experimental.pallas import tpu as pltpu
from jax.experimental.pallas import tpu_sc as plsc

N = 100000
E = 6400000
G = 64
NCLS = 10

NSC = 2
NTILE = 16
GRP = 128
WG = 16
W = GRP * WG
E_PAD = 6488064
NPAD = E_PAD - E
WINDOWS = E_PAD // (NSC * NTILE * W)
EG_ROWS = E_PAD // GRP
PWG = 6
PW = GRP * PWG
PWIN = E_PAD // (NSC * NTILE * PW)

NP1 = 100008
NACC = 100096
ACC_T = 6256
NDEG = 100096
DEG_T = 6256
SLICESZ = tuple((k * 144, 144) for k in range(43)) + ((6192, 64),)

ZROWS = 1024


def _zero_loop(zbuf, nrows, width):
    zv = jnp.zeros((16,), jnp.float32)

    def body(i, _):
        r = i // (width // 16)
        k = i % (width // 16)
        zbuf[r, pl.ds(k * 16, 16)] = zv
        return _

    lax.fori_loop(0, nrows * (width // 16), body, 0)


def _deg_body(dst2, out0, out1, acc, dstbuf, ones_v, zbuf, ssem):
    core = lax.axis_index("c")
    tile = lax.axis_index("s")

    zv = jnp.zeros((16,), jnp.float32)
    ov = jnp.ones((16,), jnp.float32)

    def stage(i, _):
        zbuf[pl.ds(i * 16, 16)] = zv
        return _

    lax.fori_loop(0, DEG_T // 16, stage, 0)

    def ones_body(i, _):
        ones_v[pl.ds(i * 16, 16)] = ov
        return _

    lax.fori_loop(0, GRP // 16, ones_body, 0)

    base = tile * DEG_T
    pltpu.sync_copy(zbuf.at[pl.ds(0, DEG_T)], acc.at[pl.ds(base, DEG_T)])
    plsc.subcore_barrier()

    gbase = (core * NTILE + tile) * (WINDOWS * WG)

    def window(w, _):
        gw = gbase + w * WG
        pltpu.sync_copy(dst2.at[pl.ds(gw, WG)], dstbuf)

        def fire(j, _):
            pltpu.async_copy(ones_v, acc.at[dstbuf.at[j]], ssem, add=True)
            return _

        lax.fori_loop(0, WG, fire, 0)

        def drain(j, _):
            pltpu.make_async_copy(ones_v, acc.at[dstbuf.at[j]], ssem).wait()
            return _

        lax.fori_loop(0, WG, drain, 0)
        return _

    lax.fori_loop(0, WINDOWS, window, 0)
    plsc.subcore_barrier()
    pltpu.sync_copy(acc.at[pl.ds(base, DEG_T)], zbuf)
    for k, o in enumerate((out0, out1)):
        @pl.when(core == k)
        def _copy(o=o):
            pltpu.sync_copy(zbuf, o.at[pl.ds(base, DEG_T)])


def _make_deg_kernel():
    mesh = plsc.VectorSubcoreMesh(core_axis_name="c", subcore_axis_name="s")
    return pl.kernel(
        _deg_body,
        out_type=[jax.ShapeDtypeStruct((NDEG,), jnp.float32),
                  jax.ShapeDtypeStruct((NDEG,), jnp.float32)],
        mesh=mesh,
        scratch_types=[
            pltpu.VMEM_SHARED((NDEG,), jnp.float32),
            pltpu.VMEM((WG, GRP), jnp.int32),
            pltpu.VMEM((GRP,), jnp.float32),
            pltpu.VMEM((DEG_T,), jnp.float32),
            pltpu.SemaphoreType.DMA,
        ],
    )


def _prop_body(nchunk, yflat, src1, dst2, out, acc, srcbuf, dstbuf, rows,
               zbuf, isem, gsem, ssem):
    core = lax.axis_index("c")
    tile = lax.axis_index("s")
    gbase = (core * NTILE + tile) * (PWIN * PWG)
    zbase = tile * ACC_T

    def issue_idx(c, w):
        eb = gbase * GRP + w * PW
        gw = gbase + w * PWG
        pltpu.async_copy(src1.at[c, pl.ds(eb, PW)],
                         srcbuf.at[lax.rem(w, 2)], isem)
        pltpu.async_copy(dst2.at[pl.ds(gw, PWG)], dstbuf.at[lax.rem(w, 3)], isem)

    def wait_idx(c, w):
        eb = gbase * GRP + w * PW
        gw = gbase + w * PWG
        pltpu.make_async_copy(src1.at[c, pl.ds(eb, PW)],
                              srcbuf.at[lax.rem(w, 2)], isem).wait()
        pltpu.make_async_copy(dst2.at[pl.ds(gw, PWG)],
                              dstbuf.at[lax.rem(w, 3)], isem).wait()

    def fire_gathers(c, w):
        b = lax.rem(w, 2)
        pltpu.async_copy(yflat.at[srcbuf.at[b]], rows.at[b], gsem)

    def drain_gathers(c, w):
        b = lax.rem(w, 2)
        pltpu.make_async_copy(yflat.at[srcbuf.at[b]], rows.at[b], gsem).wait()

    def fire_scatters(w):
        b, b3 = lax.rem(w, 2), lax.rem(w, 3)
        for j in range(PWG):
            pltpu.async_copy(rows.at[b, pl.ds(j * GRP, GRP)],
                             acc.at[dstbuf.at[b3, j]], ssem, add=True)

    def drain_scatters(w):
        b, b3 = lax.rem(w, 2), lax.rem(w, 3)
        for j in range(PWG):
            pltpu.make_async_copy(rows.at[b, pl.ds(j * GRP, GRP)],
                                  acc.at[dstbuf.at[b3, j]], ssem).wait()

    def chunk(c, _):
        _zero_loop(zbuf, 144, 16)
        for off, sz in SLICESZ:
            pltpu.sync_copy(zbuf.at[pl.ds(0, sz)], acc.at[pl.ds(zbase + off, sz)])
        plsc.subcore_barrier()

        issue_idx(c, 0)
        wait_idx(c, 0)
        fire_gathers(c, 0)
        issue_idx(c, 1)

        def window(w, _):
            @pl.when(w > 0)
            def _ds():
                drain_scatters(w - 1)

            drain_gathers(c, w)
            fire_scatters(w)

            @pl.when(w + 1 < PWIN)
            def _next():
                wait_idx(c, w + 1)
                fire_gathers(c, w + 1)

                @pl.when(w + 2 < PWIN)
                def _ii():
                    issue_idx(c, w + 2)
            return _

        lax.fori_loop(0, PWIN, window, 0)
        drain_scatters(PWIN - 1)
        plsc.subcore_barrier()
        for off, sz in SLICESZ:
            pltpu.sync_copy(acc.at[pl.ds(zbase + off, sz)],
                            zbuf.at[pl.ds(0, sz)])
            pltpu.sync_copy(zbuf.at[pl.ds(0, sz)],
                            out.at[core, pl.ds(zbase + off, sz),
                                   pl.ds(c * 16, 16)])
        plsc.subcore_barrier()
        return _

    lax.fori_loop(0, nchunk, chunk, 0)


def _make_prop_kernel(nchunk):
    mesh = plsc.VectorSubcoreMesh(core_axis_name="c", subcore_axis_name="s")
    return pl.kernel(
        functools.partial(_prop_body, nchunk),
        out_type=jax.ShapeDtypeStruct((NSC, NACC, 16 * nchunk), jnp.float32),
        name="prop%d" % nchunk,
        mesh=mesh,
        compiler_params=pltpu.CompilerParams(use_tc_tiling_on_sc=False),
        scratch_types=[
            pltpu.VMEM_SHARED((NACC, 16), jnp.float32),
            pltpu.VMEM((2, PW), jnp.int32),
            pltpu.VMEM((3, PWG, GRP), jnp.int32),
            pltpu.VMEM((2, PW, 16), jnp.float32),
            pltpu.VMEM((144, 16), jnp.float32),
            pltpu.SemaphoreType.DMA,
            pltpu.SemaphoreType.DMA,
            pltpu.SemaphoreType.DMA,
        ],
    )


PR = 8192
LR = 4096


def _prologue_body(deg0, deg1, x, dinv, y1):
    i = pl.program_id(0)
    rowid = i * PR + lax.broadcasted_iota(jnp.int32, (PR, 1), 0)
    padsub = jnp.where(rowid < NPAD, 1.0, 0.0)
    deg = deg0[...] + deg1[...] + 1.0 - padsub
    dv = lax.rsqrt(deg)
    dinv[...] = dv
    y = jnp.where(rowid < N, x[...] * dv, 0.0)
    y1[...] = jnp.concatenate(
        [y, jnp.zeros((y.shape[0], 16 - y.shape[1]), jnp.float32)], axis=1)


def _tc_prologue(deg0, deg1, x):
    grid = (pl.cdiv(NP1, PR),)
    return pl.pallas_call(
        _prologue_body,
        grid=grid,
        in_specs=[
            pl.BlockSpec((PR, 1), lambda i: (i, 0)),
            pl.BlockSpec((PR, 1), lambda i: (i, 0)),
            pl.BlockSpec((PR, 5), lambda i: (i, 0)),
        ],
        out_specs=[
            pl.BlockSpec((PR, 1), lambda i: (i, 0)),
            pl.BlockSpec((PR, 16), lambda i: (i, 0)),
        ],
        out_shape=[
            jax.ShapeDtypeStruct((NP1, 1), jnp.float32),
            jax.ShapeDtypeStruct((NP1, 16), jnp.float32),
        ],
    )(deg0, deg1, x)


def _layer_body(scale_out, accp, y, dinv, w, b, out):
    i = pl.program_id(0)
    rowid = i * LR + lax.broadcasted_iota(jnp.int32, (LR, 1), 0)
    s = (accp[0] + accp[1] + y[...]) * dinv[...]
    z = jnp.dot(s, w[...], preferred_element_type=jnp.float32) + b[...]
    h = jnp.maximum(z, 0.0)
    if scale_out:
        h = h * dinv[...]
    out[...] = jnp.where(rowid < N, h, 0.0)


def _tc_layer(accp, y, dinv, w, b, scale_out):
    k = y.shape[1]
    dout = w.shape[1]
    grid = (pl.cdiv(NP1, LR),)
    return pl.pallas_call(
        functools.partial(_layer_body, scale_out),
        grid=grid,
        in_specs=[
            pl.BlockSpec((NSC, LR, k), lambda i: (0, i, 0)),
            pl.BlockSpec((LR, k), lambda i: (i, 0)),
            pl.BlockSpec((LR, 1), lambda i: (i, 0)),
            pl.BlockSpec((k, dout), lambda i: (0, 0)),
            pl.BlockSpec((1, dout), lambda i: (0, 0)),
        ],
        out_specs=pl.BlockSpec((LR, dout), lambda i: (i, 0)),
        out_shape=jax.ShapeDtypeStruct((NP1, dout), jnp.float32),
    )(accp, y, dinv, w, b)


def _pool_body(nblk, h, batch, wfc, bfc, out, sums, counts):
    i = pl.program_id(0)

    @pl.when(i == 0)
    def _init():
        sums[...] = jnp.zeros_like(sums)
        counts[...] = jnp.zeros_like(counts)

    bb = batch[...]
    gid = lax.broadcasted_iota(jnp.int32, (LR, G), 1)
    rowid = i * LR + lax.broadcasted_iota(jnp.int32, (LR, G), 0)
    onehot = jnp.where((bb == gid) & (rowid < N), 1.0, 0.0)
    valid = (i * LR + lax.broadcasted_iota(jnp.int32, (LR, 1), 0)) < N
    hv = jnp.where(valid, h[...], 0.0)
    dn = (((0,), (0,)), ((), ()))
    sums[...] += lax.dot_general(onehot, hv, dn,
                                 preferred_element_type=jnp.float32)
    counts[...] += lax.dot_general(onehot, jnp.ones((LR, 1), jnp.float32), dn,
                                   preferred_element_type=jnp.float32)

    @pl.when(i == nblk - 1)
    def _final():
        pooled = sums[...] / jnp.maximum(counts[...], 1.0)
        z = jnp.dot(pooled, wfc[...], preferred_element_type=jnp.float32) + bfc[...]
        m = jnp.max(z, axis=1, keepdims=True)
        e = jnp.exp(z - m)
        lse = jnp.log(jnp.sum(e, axis=1, keepdims=True)) + m
        out[...] = z - lse


def _tc_poolfc(h, batch2, wfc, bfc):
    nblk = pl.cdiv(NP1, LR)
    return pl.pallas_call(
        functools.partial(_pool_body, nblk),
        grid=(nblk,),
        in_specs=[
            pl.BlockSpec((LR, 256), lambda i: (i, 0)),
            pl.BlockSpec((LR, 1), lambda i: (i, 0)),
            pl.BlockSpec((256, NCLS), lambda i: (0, 0)),
            pl.BlockSpec((1, NCLS), lambda i: (0, 0)),
        ],
        out_specs=pl.BlockSpec((G, NCLS), lambda i: (0, 0)),
        out_shape=jax.ShapeDtypeStruct((G, NCLS), jnp.float32),
        scratch_shapes=[
            pltpu.VMEM((G, 256), jnp.float32),
            pltpu.VMEM((G, 1), jnp.float32),
        ],
    )(h, batch2, wfc, bfc)


def kernel(x, edge_index, batch, W1, b1, W2, b2, W3, b3, W4, b4, Wfc, bfc):
    src = edge_index[0]
    dst = edge_index[1]
    padi = jnp.arange(NPAD, dtype=jnp.int32)
    srcp = jnp.concatenate([src, N + (padi % 8)])
    dstp = jnp.concatenate([dst, padi]).reshape(EG_ROWS, GRP)

    deg0, deg1 = _make_deg_kernel()(dstp)
    dinv, y = _tc_prologue(deg0.reshape(NDEG, 1), deg1.reshape(NDEG, 1), x)

    weights = [(jnp.pad(W1, ((0, 11), (0, 0))), b1), (W2, b2), (W3, b3), (W4, b4)]
    for li, (w, b) in enumerate(weights):
        k = w.shape[0]
        nchunk = k // 16
        yflat = y.reshape(NP1 * nchunk, 16)
        srcc = srcp[None, :] * nchunk + jnp.arange(nchunk, dtype=jnp.int32)[:, None]
        accp = _make_prop_kernel(nchunk)(yflat, srcc, dstp)
        y = _tc_layer(accp, y, dinv, w, b.reshape(1, -1), scale_out=(li < 3))

    return _tc_poolfc(y, batch.reshape(N, 1), Wfc, bfc.reshape(1, NCLS))
